# baseline (speedup 1.0000x reference)
import functools

import jax
import jax.numpy as jnp
from jax import lax
from jax.experimental import pallas as pl
from jax.experimental.pallas import tpu as pltpu

E = 8
TOPK = 2
C = 1024
H = 4096
BM = 512
BH = 512


def _router_body(x_ref, gw_ref, gb_ref, logits_ref, w_ref, idx_ref):
    lg = jnp.dot(x_ref[...], gw_ref[...], preferred_element_type=jnp.float32)
    lg = lg + gb_ref[...]
    logits_ref[...] = lg
    m = jnp.max(lg, axis=1, keepdims=True)
    p = jnp.exp(lg - m)
    p = p / jnp.sum(p, axis=1, keepdims=True)
    iota = lax.broadcasted_iota(jnp.int32, p.shape, 1)
    m1 = jnp.max(p, axis=1, keepdims=True)
    i1 = jnp.min(jnp.where(p == m1, iota, E), axis=1, keepdims=True)
    pm = jnp.where(iota == i1, -1.0, p)
    m2 = jnp.max(pm, axis=1, keepdims=True)
    i2 = jnp.min(jnp.where(pm == m2, iota, E), axis=1, keepdims=True)
    ssum = m1 + m2
    w_ref[...] = jnp.concatenate([m1 / ssum, m2 / ssum], axis=1)
    idx_ref[...] = jnp.concatenate([i1, i2], axis=1).astype(jnp.int32)


def _ffn_body(tb, te, act, fst, x_ref, w1_ref, b1_ref, w2_ref, b2_ref,
              s_ref, e_ref, out_ref):
    t = pl.program_id(0)
    h = pl.program_id(1)

    @pl.when((h == 0) & (fst[t] == 1))
    def _():
        out_ref[...] = jnp.zeros_like(out_ref)

    @pl.when(act[t] == 1)
    def _():
        xb = x_ref[...]
        hm = jnp.dot(xb, w1_ref[0], preferred_element_type=jnp.float32)
        hm = hm + b1_ref[0]
        hm = 0.5 * hm * (1.0 + lax.erf(hm * 0.7071067811865476))
        contrib = jnp.dot(hm, w2_ref[0], preferred_element_type=jnp.float32)
        scale = jnp.where(e_ref[0, 0] == te[t], s_ref[0, 0], 0.0)
        sc = scale[:, None]

        @pl.when(h == 0)
        def _():
            out_ref[...] += (contrib + b2_ref[0]) * sc

        @pl.when(h != 0)
        def _():
            out_ref[...] += contrib * sc


def kernel(x, gate_w, gate_b, fc1_w, fc1_b, fc2_w, fc2_b):
    B, L, Cd = x.shape
    N = B * L
    S = N * TOPK
    NB = S // BM
    NH = H // BH
    T = NB + E - 1

    xf = x.reshape(N, Cd)

    logits, w, idx = pl.pallas_call(
        _router_body,
        out_shape=[
            jax.ShapeDtypeStruct((N, E), jnp.float32),
            jax.ShapeDtypeStruct((N, TOPK), jnp.float32),
            jax.ShapeDtypeStruct((N, TOPK), jnp.int32),
        ],
    )(xf, gate_w, gate_b.reshape(1, E))

    idx_flat = idx.reshape(S)
    order = jnp.argsort(idx_flat, stable=True).astype(jnp.int32)
    e_sorted = jnp.take(idx_flat, order)
    tok_sorted = (order // TOPK).astype(jnp.int32)
    s_sorted = jnp.take(w.reshape(S), order)
    inv = jnp.argsort(order).astype(jnp.int32)

    counts = jnp.bincount(idx_flat, length=E).astype(jnp.int32)
    offs = jnp.concatenate(
        [jnp.zeros((1,), jnp.int32), jnp.cumsum(counts)[:-1].astype(jnp.int32)])
    ends = offs + counts
    first_b = jnp.minimum(offs // BM, NB - 1)
    last_b = jnp.where(counts > 0, jnp.maximum(ends - 1, 0) // BM, first_b)
    ntiles = (last_b - first_b + 1).astype(jnp.int32)
    tile_start = jnp.concatenate(
        [jnp.zeros((1,), jnp.int32), jnp.cumsum(ntiles)[:-1].astype(jnp.int32)])
    total = tile_start[-1] + ntiles[-1]
    t_range = jnp.arange(T, dtype=jnp.int32)
    e_of_t = jnp.clip(
        jnp.searchsorted(tile_start, t_range, side="right") - 1, 0, E - 1
    ).astype(jnp.int32)
    b_of_t = jnp.clip(first_b[e_of_t] + t_range - tile_start[e_of_t], 0, NB - 1)
    valid = t_range < total
    e_last = jnp.take(e_of_t, total - 1)
    b_last = jnp.take(b_of_t, total - 1)
    tile_e = jnp.where(valid, e_of_t, e_last).astype(jnp.int32)
    tile_b = jnp.where(valid, b_of_t, b_last).astype(jnp.int32)
    active = valid.astype(jnp.int32)
    prev_b = jnp.concatenate([jnp.full((1,), -1, jnp.int32), tile_b[:-1]])
    first = ((tile_b != prev_b) & valid).astype(jnp.int32)

    x_sorted = jnp.take(xf, tok_sorted, axis=0)

    grid_spec = pltpu.PrefetchScalarGridSpec(
        num_scalar_prefetch=4,
        grid=(T, NH),
        in_specs=[
            pl.BlockSpec((BM, C), lambda t, h, tb, te, act, fst: (tb[t], 0)),
            pl.BlockSpec((1, C, BH), lambda t, h, tb, te, act, fst: (te[t], 0, h)),
            pl.BlockSpec((1, 1, BH), lambda t, h, tb, te, act, fst: (te[t], 0, h)),
            pl.BlockSpec((1, BH, C), lambda t, h, tb, te, act, fst: (te[t], h, 0)),
            pl.BlockSpec((1, 1, C), lambda t, h, tb, te, act, fst: (te[t], 0, 0)),
            pl.BlockSpec((1, 1, BM), lambda t, h, tb, te, act, fst: (tb[t], 0, 0)),
            pl.BlockSpec((1, 1, BM), lambda t, h, tb, te, act, fst: (tb[t], 0, 0)),
        ],
        out_specs=pl.BlockSpec((BM, C), lambda t, h, tb, te, act, fst: (tb[t], 0)),
    )
    y_sorted = pl.pallas_call(
        _ffn_body,
        grid_spec=grid_spec,
        out_shape=jax.ShapeDtypeStruct((S, C), jnp.float32),
        compiler_params=pltpu.CompilerParams(
            dimension_semantics=("arbitrary", "arbitrary")),
    )(tile_b, tile_e, active, first,
      x_sorted, fc1_w, fc1_b.reshape(E, 1, H), fc2_w, fc2_b.reshape(E, 1, C),
      s_sorted.reshape(NB, 1, BM), e_sorted.reshape(NB, 1, BM))

    return y_sorted[:N].reshape(B, L, Cd) + inv[0], logits.reshape(B, L, E)
    y_pairs = jnp.take(y_sorted, inv, axis=0)
    final = y_pairs.reshape(N, TOPK, C).sum(axis=1)

    return final.reshape(B, L, Cd), logits.reshape(B, L, E)


# speedup vs baseline: 1.1479x; 1.1479x over previous
import functools

import jax
import jax.numpy as jnp
from jax import lax
from jax.experimental import pallas as pl
from jax.experimental.pallas import tpu as pltpu
from jax.experimental.pallas import tpu_sc as plsc

E = 8
TOPK = 2
C = 1024
H = 4096
BM = 512
BH = 512


def _router_body(x_ref, gw_ref, gb_ref, logits_ref, w_ref, pos_ref, cnt_ref):
    n = x_ref.shape[0]
    lg = jnp.dot(x_ref[...], gw_ref[...], preferred_element_type=jnp.float32)
    lg = lg + gb_ref[...]
    logits_ref[...] = lg
    m = jnp.max(lg, axis=1, keepdims=True)
    p = jnp.exp(lg - m)
    p = p / jnp.sum(p, axis=1, keepdims=True)
    iota = lax.broadcasted_iota(jnp.int32, p.shape, 1)
    m1 = jnp.max(p, axis=1, keepdims=True)
    i1 = jnp.min(jnp.where(p == m1, iota, E), axis=1, keepdims=True)
    pm = jnp.where(iota == i1, -1.0, p)
    m2 = jnp.max(pm, axis=1, keepdims=True)
    i2 = jnp.min(jnp.where(pm == m2, iota, E), axis=1, keepdims=True)
    ssum = m1 + m2
    w_ref[...] = jnp.concatenate([m1 / ssum, m2 / ssum], axis=1)

    a = (iota == i1).astype(jnp.float32)
    b = (iota == i2).astype(jnp.float32)
    c = a + b
    ri = lax.broadcasted_iota(jnp.int32, (n, n), 0)
    ci = lax.broadcasted_iota(jnp.int32, (n, n), 1)
    tri = (ci < ri).astype(jnp.float32)
    prefix = jnp.dot(tri, c, preferred_element_type=jnp.float32)
    counts = jnp.sum(c, axis=0, keepdims=True)
    er = lax.broadcasted_iota(jnp.int32, (E, E), 0)
    ec = lax.broadcasted_iota(jnp.int32, (E, E), 1)
    tri8 = (er < ec).astype(jnp.float32)
    c_hi = jnp.floor(counts * (1.0 / 256.0))
    c_lo = counts - 256.0 * c_hi
    offs = (256.0 * jnp.dot(c_hi, tri8, preferred_element_type=jnp.float32)
            + jnp.dot(c_lo, tri8, preferred_element_type=jnp.float32))
    base = offs + prefix
    pos0 = jnp.sum(a * base, axis=1, keepdims=True)
    pos1 = jnp.sum(b * base, axis=1, keepdims=True)
    pos_ref[...] = jnp.concatenate([pos0, pos1], axis=1).astype(jnp.int32)
    cnt_ref[...] = counts.astype(jnp.int32)


def _ffn_body(tb, te, act, fst, ofs, ens, x_ref, w1_ref, b1_ref, w2_ref,
              b2_ref, out_ref):
    t = pl.program_id(0)
    h = pl.program_id(1)

    @pl.when((h == 0) & (fst[t] == 1))
    def _():
        out_ref[...] = jnp.zeros_like(out_ref)

    @pl.when(act[t] == 1)
    def _():
        xb = x_ref[...]
        hm = jnp.dot(xb, w1_ref[0], preferred_element_type=jnp.float32)
        hm = hm + b1_ref[0]
        hm = 0.5 * hm * (1.0 + lax.erf(hm * 0.7071067811865476))
        contrib = jnp.dot(hm, w2_ref[0], preferred_element_type=jnp.float32)
        e = te[t]
        row_g = tb[t] * BM + lax.broadcasted_iota(jnp.int32, (BM, 1), 0)
        inr = (row_g >= ofs[e]) & (row_g < ens[e])
        sc = jnp.where(inr, 1.0, 0.0)

        @pl.when(h == 0)
        def _():
            out_ref[...] += (contrib + b2_ref[0]) * sc

        @pl.when(h != 0)
        def _():
            out_ref[...] += contrib * sc


def _make_dispatch(n_tok, s_rows, d, nw):
    tpw = n_tok // nw
    mesh = plsc.VectorSubcoreMesh(core_axis_name="c", subcore_axis_name="s")

    @functools.partial(
        pl.kernel, mesh=mesh,
        out_type=jax.ShapeDtypeStruct((s_rows, d), jnp.float32),
        scratch_types=[
            pltpu.VMEM((tpw, d), jnp.float32),
            pltpu.VMEM((tpw,), jnp.int32),
            pltpu.VMEM((tpw,), jnp.int32),
            pltpu.SemaphoreType.DMA,
            pltpu.SemaphoreType.DMA,
            pltpu.SemaphoreType.DMA,
        ],
    )
    def disp(xf_hbm, p0_hbm, p1_hbm, out_hbm, xbuf, p0v, p1v, s0, s1, s2):
        wid = lax.axis_index("s") * 2 + lax.axis_index("c")
        base = wid * tpw
        pltpu.sync_copy(p0_hbm.at[pl.ds(base, tpw)], p0v)
        pltpu.sync_copy(p1_hbm.at[pl.ds(base, tpw)], p1v)
        pltpu.async_copy(xf_hbm.at[pl.ds(base, tpw)], xbuf, s0).wait()
        c0 = pltpu.async_copy(xbuf, out_hbm.at[p0v], s1)
        c1 = pltpu.async_copy(xbuf, out_hbm.at[p1v], s2)
        c0.wait()
        c1.wait()

    return disp


def _make_combine(n_tok, s_rows, d, nw):
    tpw = n_tok // nw
    ch = 32
    nchunk = tpw // ch
    mesh = plsc.VectorSubcoreMesh(core_axis_name="c", subcore_axis_name="s")

    @functools.partial(
        pl.kernel, mesh=mesh,
        out_type=jax.ShapeDtypeStruct((n_tok, d), jnp.float32),
        scratch_types=[
            pltpu.VMEM((ch, d), jnp.float32),
            pltpu.VMEM((ch, d), jnp.float32),
            pltpu.VMEM((ch,), jnp.int32),
            pltpu.VMEM((ch,), jnp.int32),
            pltpu.VMEM((ch, 16), jnp.float32),
            pltpu.VMEM((ch, 16), jnp.float32),
            pltpu.SemaphoreType.DMA,
            pltpu.SemaphoreType.DMA,
        ],
    )
    def comb(y_hbm, p0_hbm, p1_hbm, w0_hbm, w1_hbm, out_hbm,
             abuf, bbuf, p0v, p1v, w0v, w1v, sa, sb):
        wid = lax.axis_index("s") * 2 + lax.axis_index("c")
        for j in range(nchunk):
            base = wid * tpw + j * ch
            pltpu.sync_copy(p0_hbm.at[pl.ds(base, ch)], p0v)
            pltpu.sync_copy(p1_hbm.at[pl.ds(base, ch)], p1v)
            ca = pltpu.async_copy(y_hbm.at[p0v], abuf, sa)
            cb = pltpu.async_copy(y_hbm.at[p1v], bbuf, sb)
            pltpu.sync_copy(w0_hbm.at[pl.ds(base, ch)], w0v)
            pltpu.sync_copy(w1_hbm.at[pl.ds(base, ch)], w1v)
            ca.wait()
            cb.wait()

            def row_fn(r, _):
                wa = w0v[r, :]
                wb = w1v[r, :]
                for cc in range(0, d, 16):
                    abuf[r, pl.ds(cc, 16)] = (
                        wa * abuf[r, pl.ds(cc, 16)] + wb * bbuf[r, pl.ds(cc, 16)])
                return 0

            lax.fori_loop(0, ch, row_fn, 0)
            pltpu.sync_copy(abuf, out_hbm.at[pl.ds(base, ch)])

    return comb


def kernel(x, gate_w, gate_b, fc1_w, fc1_b, fc2_w, fc2_b):
    B, L, Cd = x.shape
    N = B * L
    S = N * TOPK
    NB = S // BM
    NH = H // BH
    T = NB + E - 1

    info = plsc.get_sparse_core_info()
    NW = info.num_cores * info.num_subcores

    xf = x.reshape(N, Cd)

    logits, w, pos, cnt = pl.pallas_call(
        _router_body,
        out_shape=[
            jax.ShapeDtypeStruct((N, E), jnp.float32),
            jax.ShapeDtypeStruct((N, TOPK), jnp.float32),
            jax.ShapeDtypeStruct((N, TOPK), jnp.int32),
            jax.ShapeDtypeStruct((1, E), jnp.int32),
        ],
    )(xf, gate_w, gate_b.reshape(1, E))

    counts = cnt[0]
    offs = jnp.concatenate(
        [jnp.zeros((1,), jnp.int32), jnp.cumsum(counts)[:-1].astype(jnp.int32)])
    ends = offs + counts
    first_b = jnp.minimum(offs // BM, NB - 1)
    last_b = jnp.where(counts > 0, jnp.maximum(ends - 1, 0) // BM, first_b)
    ntiles = (last_b - first_b + 1).astype(jnp.int32)
    tile_start = jnp.concatenate(
        [jnp.zeros((1,), jnp.int32), jnp.cumsum(ntiles)[:-1].astype(jnp.int32)])
    total = tile_start[-1] + ntiles[-1]
    t_range = jnp.arange(T, dtype=jnp.int32)
    e_of_t = jnp.clip(
        jnp.searchsorted(tile_start, t_range, side="right") - 1, 0, E - 1
    ).astype(jnp.int32)
    b_of_t = jnp.clip(first_b[e_of_t] + t_range - tile_start[e_of_t], 0, NB - 1)
    valid = t_range < total
    e_last = jnp.take(e_of_t, total - 1)
    b_last = jnp.take(b_of_t, total - 1)
    tile_e = jnp.where(valid, e_of_t, e_last).astype(jnp.int32)
    tile_b = jnp.where(valid, b_of_t, b_last).astype(jnp.int32)
    active = valid.astype(jnp.int32)
    prev_b = jnp.concatenate([jnp.full((1,), -1, jnp.int32), tile_b[:-1]])
    first = ((tile_b != prev_b) & valid).astype(jnp.int32)

    pos0 = pos[:, 0]
    pos1 = pos[:, 1]
    x_sorted = _make_dispatch(N, S, Cd, NW)(xf, pos0, pos1)

    grid_spec = pltpu.PrefetchScalarGridSpec(
        num_scalar_prefetch=6,
        grid=(T, NH),
        in_specs=[
            pl.BlockSpec((BM, C), lambda t, h, tb, te, a_, f_, o_, n_: (tb[t], 0)),
            pl.BlockSpec((1, C, BH), lambda t, h, tb, te, a_, f_, o_, n_: (te[t], 0, h)),
            pl.BlockSpec((1, 1, BH), lambda t, h, tb, te, a_, f_, o_, n_: (te[t], 0, h)),
            pl.BlockSpec((1, BH, C), lambda t, h, tb, te, a_, f_, o_, n_: (te[t], h, 0)),
            pl.BlockSpec((1, 1, C), lambda t, h, tb, te, a_, f_, o_, n_: (te[t], 0, 0)),
        ],
        out_specs=pl.BlockSpec((BM, C), lambda t, h, tb, te, a_, f_, o_, n_: (tb[t], 0)),
    )
    y_sorted = pl.pallas_call(
        _ffn_body,
        grid_spec=grid_spec,
        out_shape=jax.ShapeDtypeStruct((S, C), jnp.float32),
        compiler_params=pltpu.CompilerParams(
            dimension_semantics=("arbitrary", "arbitrary")),
    )(tile_b, tile_e, active, first, offs, ends,
      x_sorted, fc1_w, fc1_b.reshape(E, 1, H), fc2_w, fc2_b.reshape(E, 1, C))

    w0b = jnp.broadcast_to(w[:, 0:1], (N, 16))
    w1b = jnp.broadcast_to(w[:, 1:2], (N, 16))
    final = _make_combine(N, S, Cd, NW)(y_sorted, pos0, pos1, w0b, w1b)

    return final.reshape(B, L, Cd), logits.reshape(B, L, E)
